# baseline (speedup 1.0000x reference)
import functools

import jax
import jax.numpy as jnp
import numpy as np
from jax import lax
from jax.experimental import pallas as pl
from jax.experimental.pallas import tpu as pltpu
from jax.experimental.pallas import tpu_sc as plsc

NC = 2
NS = 16
NW = NC * NS
WIN = 128


def _tc_prep(xin, W_src, Wl, bl, a_src, a_dst, W_dst, bm):
    n, d = xin.shape

    def body(x_ref, ws_ref, wl_ref, bl_ref, as_ref, ad_ref, wd_ref,
             h_ref, sk_ref, es_ref, ed_ref):
        xb = x_ref[...]
        h_ref[...] = jnp.dot(xb, ws_ref[...], preferred_element_type=jnp.float32)
        sk_ref[...] = jnp.dot(xb, wl_ref[...], preferred_element_type=jnp.float32) + bl_ref[...]
        ws2 = jnp.sum(ws_ref[...] * as_ref[...], axis=1, keepdims=True)
        wd2 = jnp.sum(wd_ref[...] * ad_ref[...], axis=1, keepdims=True)
        es_ref[...] = jnp.dot(xb, ws2, preferred_element_type=jnp.float32)
        ed_ref[...] = jnp.dot(xb, wd2, preferred_element_type=jnp.float32)

    full = lambda i: (0, 0)
    return pl.pallas_call(
        body,
        grid=(pl.cdiv(n, bm),),
        in_specs=[pl.BlockSpec((bm, d), lambda i: (i, 0)),
                  pl.BlockSpec((d, d), full),
                  pl.BlockSpec((d, d), full),
                  pl.BlockSpec((1, d), full),
                  pl.BlockSpec((1, d), full),
                  pl.BlockSpec((1, d), full),
                  pl.BlockSpec((d, d), full)],
        out_specs=[pl.BlockSpec((bm, d), lambda i: (i, 0)),
                   pl.BlockSpec((bm, d), lambda i: (i, 0)),
                   pl.BlockSpec((bm, 1), lambda i: (i, 0)),
                   pl.BlockSpec((bm, 1), lambda i: (i, 0))],
        out_shape=[jax.ShapeDtypeStruct((n, d), jnp.float32),
                   jax.ShapeDtypeStruct((n, d), jnp.float32),
                   jax.ShapeDtypeStruct((n, 1), jnp.float32),
                   jax.ShapeDtypeStruct((n, 1), jnp.float32)],
    )(xin, W_src, Wl, bl.reshape(1, d), a_src.reshape(1, d),
      a_dst.reshape(1, d), W_dst)


def _tc_fuse_prep(part, b, skip, W_src, Wl, bl, a_src, a_dst, W_dst, bm):
    _, n, d = part.shape

    def body(p_ref, b_ref, skA_ref, ws_ref, wl_ref, bl_ref, as_ref, ad_ref,
             wd_ref, h2_ref, sk_ref, es_ref, ed_ref):
        hb = jnp.maximum(p_ref[0] + p_ref[1] + b_ref[...] + skA_ref[...], 0.0)
        h2_ref[...] = jnp.dot(hb, ws_ref[...], preferred_element_type=jnp.float32)
        sk_ref[...] = jnp.dot(hb, wl_ref[...], preferred_element_type=jnp.float32) + bl_ref[...]
        ws2 = jnp.sum(ws_ref[...] * as_ref[...], axis=1, keepdims=True)
        wd2 = jnp.sum(wd_ref[...] * ad_ref[...], axis=1, keepdims=True)
        es_ref[...] = jnp.dot(hb, ws2, preferred_element_type=jnp.float32)
        ed_ref[...] = jnp.dot(hb, wd2, preferred_element_type=jnp.float32)

    full = lambda i: (0, 0)
    return pl.pallas_call(
        body,
        grid=(pl.cdiv(n, bm),),
        in_specs=[pl.BlockSpec((2, bm, d), lambda i: (0, i, 0)),
                  pl.BlockSpec((1, d), full),
                  pl.BlockSpec((bm, d), lambda i: (i, 0)),
                  pl.BlockSpec((d, d), full),
                  pl.BlockSpec((d, d), full),
                  pl.BlockSpec((1, d), full),
                  pl.BlockSpec((1, d), full),
                  pl.BlockSpec((1, d), full),
                  pl.BlockSpec((d, d), full)],
        out_specs=[pl.BlockSpec((bm, d), lambda i: (i, 0)),
                   pl.BlockSpec((bm, d), lambda i: (i, 0)),
                   pl.BlockSpec((bm, 1), lambda i: (i, 0)),
                   pl.BlockSpec((bm, 1), lambda i: (i, 0))],
        out_shape=[jax.ShapeDtypeStruct((n, d), jnp.float32),
                   jax.ShapeDtypeStruct((n, d), jnp.float32),
                   jax.ShapeDtypeStruct((n, 1), jnp.float32),
                   jax.ShapeDtypeStruct((n, 1), jnp.float32)],
    )(part, b.reshape(1, d), skip, W_src, Wl, bl.reshape(1, d),
      a_src.reshape(1, d), a_dst.reshape(1, d), W_dst)


def _tc_final(part, b, skip, bm):
    _, n, d = part.shape

    def body(p_ref, b_ref, sk_ref, o_ref):
        o_ref[...] = p_ref[0] + p_ref[1] + b_ref[...] + sk_ref[...]

    full = lambda i: (0, 0)
    return pl.pallas_call(
        body,
        grid=(pl.cdiv(n, bm),),
        in_specs=[pl.BlockSpec((2, bm, d), lambda i: (0, i, 0)),
                  pl.BlockSpec((1, d), full),
                  pl.BlockSpec((bm, d), lambda i: (i, 0))],
        out_specs=pl.BlockSpec((bm, d), lambda i: (i, 0)),
        out_shape=jax.ShapeDtypeStruct((n, d), jnp.float32),
    )(part, b.reshape(1, d), skip)


SB = 8


def _sc_edge(es_flat, ed_flat, hw, src3, dst3, n_nodes, per, d):
    wn = src3.shape[1]
    nsb = wn // SB
    assert wn % SB == 0
    npt = ((n_nodes // NS) // 8) * 8
    tail = n_nodes - NS * npt
    assert tail <= WIN and npt % 16 == 0 and tail % 16 == 0
    chunks = [WIN] * (npt // WIN) + ([npt % WIN] if npt % WIN else [])

    mesh = plsc.VectorSubcoreMesh(core_axis_name="c", subcore_axis_name="s",
                                  num_cores=NC, num_subcores=NS)

    @functools.partial(
        pl.kernel,
        out_type=jax.ShapeDtypeStruct((NC, n_nodes, d), jnp.float32),
        mesh=mesh,
        compiler_params=pltpu.CompilerParams(needs_layout_passes=False),
        scratch_types=[
            pltpu.VMEM((SB, WIN), jnp.int32),
            pltpu.VMEM((SB, WIN), jnp.int32),
            pltpu.VMEM((WIN, d), jnp.float32),
            pltpu.VMEM((WIN,), jnp.float32),
            pltpu.VMEM((WIN,), jnp.float32),
            pltpu.VMEM((WIN,), jnp.float32),
            pltpu.VMEM((npt + 16,), jnp.float32),
            pltpu.VMEM((WIN, d), jnp.float32),
            pltpu.VMEM_SHARED((n_nodes,), jnp.float32),
            pltpu.VMEM_SHARED((n_nodes,), jnp.float32),
            pltpu.VMEM_SHARED((n_nodes,), jnp.float32),
            pltpu.VMEM_SHARED((n_nodes, d), jnp.float32),
            pltpu.SemaphoreType.DMA,
            pltpu.SemaphoreType.DMA,
            pltpu.SemaphoreType.DMA,
        ],
    )
    def ek(es_hbm, ed_hbm, hw_hbm, src_hbm, dst_hbm, part_hbm,
           srcBlk, dstBlk, rowsW, esg, edg, exw, invb, rows,
           es_sh, ed_sh, s_sh, out_sh, sem, semB, semS):
        c = lax.axis_index("c")
        s = lax.axis_index("s")
        own = c * NS + s
        mir = (1 - c) * NS + s
        nb = s * npt
        last = NS - 1
        zv = jnp.zeros((16,), jnp.float32)

        def stage_blk(chunk, sbi):
            pltpu.sync_copy(src_hbm.at[chunk].at[pl.ds(sbi * SB, SB)], srcBlk)
            pltpu.sync_copy(dst_hbm.at[chunk].at[pl.ds(sbi * SB, SB)], dstBlk)

        pltpu.sync_copy(es_hbm.at[pl.ds(nb, npt)], invb.at[pl.ds(0, npt)])
        pltpu.sync_copy(invb.at[pl.ds(0, npt)], es_sh.at[pl.ds(nb, npt)])
        pltpu.sync_copy(ed_hbm.at[pl.ds(nb, npt)], invb.at[pl.ds(0, npt)])
        pltpu.sync_copy(invb.at[pl.ds(0, npt)], ed_sh.at[pl.ds(nb, npt)])

        @pl.when(s == last)
        def _():
            tb = NS * npt
            pltpu.sync_copy(es_hbm.at[pl.ds(tb, tail)], exw.at[pl.ds(0, tail)])
            pltpu.sync_copy(exw.at[pl.ds(0, tail)], es_sh.at[pl.ds(tb, tail)])
            pltpu.sync_copy(ed_hbm.at[pl.ds(tb, tail)], exw.at[pl.ds(0, tail)])
            pltpu.sync_copy(exw.at[pl.ds(0, tail)], ed_sh.at[pl.ds(tb, tail)])

        def zs(i, _):
            invb[pl.ds(16 * i, 16)] = zv
            return 0
        lax.fori_loop(0, (npt + 16) // 16, zs, 0)
        pltpu.sync_copy(invb.at[pl.ds(0, npt)], s_sh.at[pl.ds(nb, npt)])

        @pl.when(s == last)
        def _():
            pltpu.sync_copy(invb.at[pl.ds(0, tail)],
                            s_sh.at[pl.ds(NS * npt, tail)])

        def zrow(i, _):
            for j in range(d // 16):
                rows[i, pl.ds(16 * j, 16)] = zv
            return 0
        lax.fori_loop(0, WIN, zrow, 0)
        off = 0
        for cl in chunks:
            pltpu.sync_copy(rows.at[pl.ds(0, cl)],
                            out_sh.at[pl.ds(nb + off, cl)])
            off += cl

        @pl.when(s == last)
        def _():
            pltpu.sync_copy(rows.at[pl.ds(0, tail)],
                            out_sh.at[pl.ds(NS * npt, tail)])

        plsc.subcore_barrier()

        def exp_vec(w, v):
            e = esg[pl.ds(16 * v, 16)] + edg[pl.ds(16 * v, 16)]
            e = jnp.where(e > 0, e, 0.2 * e)
            e = jnp.minimum(e, 70.0)
            ex = jnp.exp(e)
            pos = w * WIN + 16 * v + lax.iota(jnp.int32, 16)
            return jnp.where(pos < per, ex, 0.0)

        def drain_scalar():
            pltpu.make_async_copy(es_hbm.at[pl.ds(0, WIN)], exw, semS).wait()

        def drain_rows():
            pltpu.make_async_copy(part_hbm.at[0, pl.ds(0, WIN)], rows,
                                  semS).wait()

        def sb1(chunk):
            def body(sbi, _):
                @pl.when(sbi > 0)
                def _():
                    drain_scalar()
                stage_blk(chunk, sbi)
                for j in range(SB):
                    w = sbi * SB + j
                    ga = pltpu.async_copy(es_sh.at[srcBlk.at[j]], esg, sem)
                    gb = pltpu.async_copy(ed_sh.at[dstBlk.at[j]], edg, semB)
                    ga.wait()
                    gb.wait()
                    if j > 0:
                        drain_scalar()
                    for v in range(WIN // 16):
                        exw[pl.ds(16 * v, 16)] = exp_vec(w, v)
                    pltpu.async_copy(exw, s_sh.at[dstBlk.at[j]],
                                     semS, add=True)
                return 0
            lax.fori_loop(0, nsb, body, 0)
            drain_scalar()

        sb1(own)
        sb1(mir)

        plsc.subcore_barrier()

        pltpu.sync_copy(s_sh.at[pl.ds(nb, npt)], invb.at[pl.ds(0, npt)])

        def inv(i, _):
            sv = invb[pl.ds(16 * i, 16)]
            invb[pl.ds(16 * i, 16)] = 1.0 / (sv + 1e-16)
            return 0
        lax.fori_loop(0, npt // 16, inv, 0)
        pltpu.sync_copy(invb.at[pl.ds(0, npt)], s_sh.at[pl.ds(nb, npt)])

        @pl.when(s == last)
        def _():
            tb = NS * npt
            pltpu.sync_copy(s_sh.at[pl.ds(tb, tail)], invb.at[pl.ds(0, tail)])
            sv = invb[pl.ds(0, 16)]
            invb[pl.ds(0, 16)] = 1.0 / (sv + 1e-16)
            pltpu.sync_copy(invb.at[pl.ds(0, tail)], s_sh.at[pl.ds(tb, tail)])

        plsc.subcore_barrier()

        def sb2(sbi, _):
            @pl.when(sbi > 0)
            def _():
                drain_rows()
            stage_blk(own, sbi)
            for j in range(SB):
                w = sbi * SB + j
                g = pltpu.async_copy(hw_hbm.at[srcBlk.at[j]], rowsW, sem)
                pltpu.sync_copy(es_sh.at[srcBlk.at[j]], esg)
                pltpu.sync_copy(ed_sh.at[dstBlk.at[j]], edg)
                pltpu.sync_copy(s_sh.at[dstBlk.at[j]], exw)
                for v in range(WIN // 16):
                    edg[pl.ds(16 * v, 16)] = (exp_vec(w, v)
                                              * exw[pl.ds(16 * v, 16)])
                if j > 0:
                    drain_rows()
                g.wait()

                def scale_v(v, _):
                    av = edg[pl.ds(16 * v, 16)]
                    for t in range(16):
                        a = av[t]
                        i = 16 * v + t
                        for jj in range(d // 16):
                            rows[i, pl.ds(16 * jj, 16)] = (
                                rowsW[i, pl.ds(16 * jj, 16)] * a)
                    return 0
                lax.fori_loop(0, WIN // 16, scale_v, 0)
                pltpu.async_copy(rows, out_sh.at[dstBlk.at[j]], semS, add=True)
            return 0
        lax.fori_loop(0, nsb, sb2, 0)
        drain_rows()

        plsc.subcore_barrier()

        off = 0
        for cl in chunks:
            pltpu.sync_copy(out_sh.at[pl.ds(nb + off, cl)],
                            rows.at[pl.ds(0, cl)])
            pltpu.sync_copy(rows.at[pl.ds(0, cl)],
                            part_hbm.at[c, pl.ds(nb + off, cl)])
            off += cl

        @pl.when(s == last)
        def _():
            tb = NS * npt
            pltpu.sync_copy(out_sh.at[pl.ds(tb, tail)], rows.at[pl.ds(0, tail)])
            pltpu.sync_copy(rows.at[pl.ds(0, tail)],
                            part_hbm.at[c, pl.ds(tb, tail)])

    return ek(es_flat, ed_flat, hw, src3, dst3)


def kernel(x, edge_index, W1_src, W1_dst, a1_src, a1_dst, b1, Wl1, bl1,
           W2_src, W2_dst, a2_src, a2_dst, b2, Wl2, bl2):
    n, d = x.shape
    e = edge_index.shape[1]
    per = e // NW
    wn = pl.cdiv(pl.cdiv(per, WIN), SB) * SB
    perp = wn * WIN

    src = edge_index[0].reshape(NW, per)
    dst = edge_index[1].reshape(NW, per)
    pad = jnp.zeros((NW, perp - per), jnp.int32)
    src3 = jnp.concatenate([src, pad], axis=1).reshape(NW, wn, WIN)
    dst3 = jnp.concatenate([dst, pad], axis=1).reshape(NW, wn, WIN)

    bm = 512

    h1, skip1, es1, ed1 = _tc_prep(x, W1_src, Wl1, bl1, a1_src,
                                   a1_dst, W1_dst, bm)
    part1 = _sc_edge(es1.reshape(n), ed1.reshape(n), h1,
                     src3, dst3, n, per, d)
    h2, skip2, es2, ed2 = _tc_fuse_prep(part1, b1, skip1, W2_src, Wl2,
                                        bl2, a2_src, a2_dst, W2_dst, bm)
    part2 = _sc_edge(es2.reshape(n), ed2.reshape(n), h2,
                     src3, dst3, n, per, d)
    return _tc_final(part2, b2, skip2, bm)


# speedup vs baseline: 1.4762x; 1.4762x over previous
import functools

import jax
import jax.numpy as jnp
from jax import lax
from jax.experimental import pallas as pl
from jax.experimental.pallas import tpu as pltpu
from jax.experimental.pallas import tpu_sc as plsc

NC = 2
NS = 16
NW = NC * NS
WIN = 128


def _tc_prep(xin, W_src, Wl, bl, a_src, a_dst, W_dst, bm):
    n, d = xin.shape

    def body(x_ref, ws_ref, wl_ref, bl_ref, as_ref, ad_ref, wd_ref,
             h_ref, sk_ref, es_ref, ed_ref):
        xb = x_ref[...]
        h_ref[...] = jnp.dot(xb, ws_ref[...], preferred_element_type=jnp.float32)
        sk_ref[...] = jnp.dot(xb, wl_ref[...], preferred_element_type=jnp.float32) + bl_ref[...]
        ws2 = jnp.sum(ws_ref[...] * as_ref[...], axis=1, keepdims=True)
        wd2 = jnp.sum(wd_ref[...] * ad_ref[...], axis=1, keepdims=True)
        es_ref[...] = jnp.dot(xb, ws2, preferred_element_type=jnp.float32)
        ed_ref[...] = jnp.dot(xb, wd2, preferred_element_type=jnp.float32)

    full = lambda i: (0, 0)
    return pl.pallas_call(
        body,
        grid=(pl.cdiv(n, bm),),
        in_specs=[pl.BlockSpec((bm, d), lambda i: (i, 0)),
                  pl.BlockSpec((d, d), full),
                  pl.BlockSpec((d, d), full),
                  pl.BlockSpec((1, d), full),
                  pl.BlockSpec((1, d), full),
                  pl.BlockSpec((1, d), full),
                  pl.BlockSpec((d, d), full)],
        out_specs=[pl.BlockSpec((bm, d), lambda i: (i, 0)),
                   pl.BlockSpec((bm, d), lambda i: (i, 0)),
                   pl.BlockSpec((bm, 1), lambda i: (i, 0)),
                   pl.BlockSpec((bm, 1), lambda i: (i, 0))],
        out_shape=[jax.ShapeDtypeStruct((n, d), jnp.float32),
                   jax.ShapeDtypeStruct((n, d), jnp.float32),
                   jax.ShapeDtypeStruct((n, 1), jnp.float32),
                   jax.ShapeDtypeStruct((n, 1), jnp.float32)],
    )(xin, W_src, Wl, bl.reshape(1, d), a_src.reshape(1, d),
      a_dst.reshape(1, d), W_dst)


def _tc_fuse_prep(part, b, skip, W_src, Wl, bl, a_src, a_dst, W_dst, bm):
    _, n, d = part.shape

    def body(p_ref, b_ref, skA_ref, ws_ref, wl_ref, bl_ref, as_ref, ad_ref,
             wd_ref, h2_ref, sk_ref, es_ref, ed_ref):
        hb = jnp.maximum(p_ref[0] + p_ref[1] + b_ref[...] + skA_ref[...], 0.0)
        h2_ref[...] = jnp.dot(hb, ws_ref[...], preferred_element_type=jnp.float32)
        sk_ref[...] = jnp.dot(hb, wl_ref[...], preferred_element_type=jnp.float32) + bl_ref[...]
        ws2 = jnp.sum(ws_ref[...] * as_ref[...], axis=1, keepdims=True)
        wd2 = jnp.sum(wd_ref[...] * ad_ref[...], axis=1, keepdims=True)
        es_ref[...] = jnp.dot(hb, ws2, preferred_element_type=jnp.float32)
        ed_ref[...] = jnp.dot(hb, wd2, preferred_element_type=jnp.float32)

    full = lambda i: (0, 0)
    return pl.pallas_call(
        body,
        grid=(pl.cdiv(n, bm),),
        in_specs=[pl.BlockSpec((2, bm, d), lambda i: (0, i, 0)),
                  pl.BlockSpec((1, d), full),
                  pl.BlockSpec((bm, d), lambda i: (i, 0)),
                  pl.BlockSpec((d, d), full),
                  pl.BlockSpec((d, d), full),
                  pl.BlockSpec((1, d), full),
                  pl.BlockSpec((1, d), full),
                  pl.BlockSpec((1, d), full),
                  pl.BlockSpec((d, d), full)],
        out_specs=[pl.BlockSpec((bm, d), lambda i: (i, 0)),
                   pl.BlockSpec((bm, d), lambda i: (i, 0)),
                   pl.BlockSpec((bm, 1), lambda i: (i, 0)),
                   pl.BlockSpec((bm, 1), lambda i: (i, 0))],
        out_shape=[jax.ShapeDtypeStruct((n, d), jnp.float32),
                   jax.ShapeDtypeStruct((n, d), jnp.float32),
                   jax.ShapeDtypeStruct((n, 1), jnp.float32),
                   jax.ShapeDtypeStruct((n, 1), jnp.float32)],
    )(part, b.reshape(1, d), skip, W_src, Wl, bl.reshape(1, d),
      a_src.reshape(1, d), a_dst.reshape(1, d), W_dst)


def _tc_final(part, b, skip, bm):
    _, n, d = part.shape

    def body(p_ref, b_ref, sk_ref, o_ref):
        o_ref[...] = p_ref[0] + p_ref[1] + b_ref[...] + sk_ref[...]

    full = lambda i: (0, 0)
    return pl.pallas_call(
        body,
        grid=(pl.cdiv(n, bm),),
        in_specs=[pl.BlockSpec((2, bm, d), lambda i: (0, i, 0)),
                  pl.BlockSpec((1, d), full),
                  pl.BlockSpec((bm, d), lambda i: (i, 0))],
        out_specs=pl.BlockSpec((bm, d), lambda i: (i, 0)),
        out_shape=jax.ShapeDtypeStruct((n, d), jnp.float32),
    )(part, b.reshape(1, d), skip)


def _sc_edge(es_flat, ed_flat, h, src3, dst3, n_nodes, per, d):
    wn = src3.shape[1]
    npt = ((n_nodes // NS) // 8) * 8
    tail = n_nodes - NS * npt
    assert tail <= WIN and npt % 16 == 0 and tail % 16 == 0
    chunks = [WIN] * (npt // WIN) + ([npt % WIN] if npt % WIN else [])

    mesh = plsc.VectorSubcoreMesh(core_axis_name="c", subcore_axis_name="s",
                                  num_cores=NC, num_subcores=NS)

    @functools.partial(
        pl.kernel,
        out_type=jax.ShapeDtypeStruct((NC, n_nodes, d), jnp.float32),
        mesh=mesh,
        compiler_params=pltpu.CompilerParams(needs_layout_passes=False),
        scratch_types=[
            pltpu.VMEM((wn, WIN), jnp.int32),
            pltpu.VMEM((wn, WIN), jnp.int32),
            pltpu.VMEM((wn, WIN), jnp.float32),
            pltpu.VMEM((WIN,), jnp.float32),
            pltpu.VMEM((WIN,), jnp.float32),
            pltpu.VMEM((WIN,), jnp.float32),
            pltpu.VMEM((npt + 16,), jnp.float32),
            pltpu.VMEM((WIN, d), jnp.float32),
            pltpu.VMEM_SHARED((n_nodes,), jnp.float32),
            pltpu.VMEM_SHARED((n_nodes,), jnp.float32),
            pltpu.VMEM_SHARED((n_nodes,), jnp.float32),
            pltpu.VMEM_SHARED((n_nodes, d), jnp.float32),
            pltpu.SemaphoreType.DMA,
            pltpu.SemaphoreType.DMA,
            pltpu.SemaphoreType.DMA,
        ],
    )
    def ek(es_hbm, ed_hbm, h_hbm, src_hbm, dst_hbm, part_hbm,
           srcO, dstO, ex2d, esg, edg, exw, invb, rows,
           es_sh, ed_sh, s_sh, out_sh, sem, semB, semS):
        c = lax.axis_index("c")
        s = lax.axis_index("s")
        own = c * NS + s
        mir = (1 - c) * NS + s
        nb = s * npt
        last = NS - 1
        zv = jnp.zeros((16,), jnp.float32)

        pltpu.sync_copy(src_hbm.at[own], srcO)
        pltpu.sync_copy(dst_hbm.at[own], dstO)

        pltpu.sync_copy(es_hbm.at[pl.ds(nb, npt)], invb.at[pl.ds(0, npt)])
        pltpu.sync_copy(invb.at[pl.ds(0, npt)], es_sh.at[pl.ds(nb, npt)])
        pltpu.sync_copy(ed_hbm.at[pl.ds(nb, npt)], invb.at[pl.ds(0, npt)])
        pltpu.sync_copy(invb.at[pl.ds(0, npt)], ed_sh.at[pl.ds(nb, npt)])

        @pl.when(s == last)
        def _():
            tb = NS * npt
            pltpu.sync_copy(es_hbm.at[pl.ds(tb, tail)], exw.at[pl.ds(0, tail)])
            pltpu.sync_copy(exw.at[pl.ds(0, tail)], es_sh.at[pl.ds(tb, tail)])
            pltpu.sync_copy(ed_hbm.at[pl.ds(tb, tail)], exw.at[pl.ds(0, tail)])
            pltpu.sync_copy(exw.at[pl.ds(0, tail)], ed_sh.at[pl.ds(tb, tail)])

        def zs(i, _):
            invb[pl.ds(16 * i, 16)] = zv
            return 0
        lax.fori_loop(0, (npt + 16) // 16, zs, 0)
        pltpu.sync_copy(invb.at[pl.ds(0, npt)], s_sh.at[pl.ds(nb, npt)])

        @pl.when(s == last)
        def _():
            pltpu.sync_copy(invb.at[pl.ds(0, tail)],
                            s_sh.at[pl.ds(NS * npt, tail)])

        def zrow(i, _):
            for j in range(d // 16):
                rows[i, pl.ds(16 * j, 16)] = zv
            return 0
        lax.fori_loop(0, WIN, zrow, 0)
        off = 0
        for cl in chunks:
            pltpu.sync_copy(rows.at[pl.ds(0, cl)],
                            out_sh.at[pl.ds(nb + off, cl)])
            off += cl

        @pl.when(s == last)
        def _():
            pltpu.sync_copy(rows.at[pl.ds(0, tail)],
                            out_sh.at[pl.ds(NS * npt, tail)])

        plsc.subcore_barrier()

        def exp_win(w, out_ref, keep2d):
            for v in range(WIN // 16):
                e = esg[pl.ds(16 * v, 16)] + edg[pl.ds(16 * v, 16)]
                e = jnp.where(e > 0, e, 0.2 * e)
                e = jnp.minimum(e, 70.0)
                ex = jnp.exp(e)
                pos = w * WIN + 16 * v + lax.iota(jnp.int32, 16)
                ex = jnp.where(pos < per, ex, 0.0)
                if keep2d:
                    out_ref[w, pl.ds(16 * v, 16)] = ex
                else:
                    out_ref[pl.ds(16 * v, 16)] = ex

        def drain_scalar():
            pltpu.make_async_copy(es_hbm.at[pl.ds(0, WIN)], exw, semS).wait()

        def drain_rows():
            pltpu.make_async_copy(h_hbm.at[pl.ds(0, WIN)], rows, semS).wait()

        def win1(w, _):
            ga = pltpu.async_copy(es_sh.at[srcO.at[w]], esg, sem)
            gb = pltpu.async_copy(ed_sh.at[dstO.at[w]], edg, semB)
            ga.wait()
            gb.wait()
            exp_win(w, ex2d, True)

            @pl.when(w > 0)
            def _():
                drain_scalar()
            pltpu.async_copy(ex2d.at[w], s_sh.at[dstO.at[w]], semS, add=True)
            return 0
        lax.fori_loop(0, wn, win1, 0)
        drain_scalar()

        pltpu.sync_copy(src_hbm.at[mir], srcO)
        pltpu.sync_copy(dst_hbm.at[mir], dstO)

        def win1m(w, _):
            ga = pltpu.async_copy(es_sh.at[srcO.at[w]], esg, sem)
            gb = pltpu.async_copy(ed_sh.at[dstO.at[w]], edg, semB)
            ga.wait()
            gb.wait()

            @pl.when(w > 0)
            def _():
                drain_scalar()
            exp_win(w, exw, False)
            pltpu.async_copy(exw, s_sh.at[dstO.at[w]], semS, add=True)
            return 0
        lax.fori_loop(0, wn, win1m, 0)
        drain_scalar()

        plsc.subcore_barrier()

        pltpu.sync_copy(s_sh.at[pl.ds(nb, npt)], invb.at[pl.ds(0, npt)])

        def inv(i, _):
            sv = invb[pl.ds(16 * i, 16)]
            invb[pl.ds(16 * i, 16)] = 1.0 / (sv + 1e-16)
            return 0
        lax.fori_loop(0, npt // 16, inv, 0)
        pltpu.sync_copy(invb.at[pl.ds(0, npt)], s_sh.at[pl.ds(nb, npt)])

        @pl.when(s == last)
        def _():
            tb = NS * npt
            pltpu.sync_copy(s_sh.at[pl.ds(tb, tail)], invb.at[pl.ds(0, tail)])
            sv = invb[pl.ds(0, 16)]
            invb[pl.ds(0, 16)] = 1.0 / (sv + 1e-16)
            pltpu.sync_copy(invb.at[pl.ds(0, tail)], s_sh.at[pl.ds(tb, tail)])

        pltpu.sync_copy(src_hbm.at[own], srcO)
        pltpu.sync_copy(dst_hbm.at[own], dstO)
        plsc.subcore_barrier()

        def win2(w, _):
            pltpu.sync_copy(s_sh.at[dstO.at[w]], edg)
            for v in range(WIN // 16):
                edg[pl.ds(16 * v, 16)] = (ex2d[w, pl.ds(16 * v, 16)]
                                          * edg[pl.ds(16 * v, 16)])

            @pl.when(w > 0)
            def _():
                drain_rows()
            pltpu.async_copy(h_hbm.at[srcO.at[w]], rows, sem).wait()

            def scale_v(v, _):
                av = edg[pl.ds(16 * v, 16)]
                for t in range(16):
                    a = av[t]
                    i = 16 * v + t
                    for j in range(d // 16):
                        rows[i, pl.ds(16 * j, 16)] = rows[i, pl.ds(16 * j, 16)] * a
                return 0
            lax.fori_loop(0, WIN // 16, scale_v, 0, unroll=2)
            pltpu.async_copy(rows, out_sh.at[dstO.at[w]], semS, add=True)
            return 0
        lax.fori_loop(0, wn, win2, 0)
        drain_rows()

        plsc.subcore_barrier()

        off = 0
        for cl in chunks:
            pltpu.sync_copy(out_sh.at[pl.ds(nb + off, cl)],
                            rows.at[pl.ds(0, cl)])
            pltpu.sync_copy(rows.at[pl.ds(0, cl)],
                            part_hbm.at[c, pl.ds(nb + off, cl)])
            off += cl

        @pl.when(s == last)
        def _():
            tb = NS * npt
            pltpu.sync_copy(out_sh.at[pl.ds(tb, tail)], rows.at[pl.ds(0, tail)])
            pltpu.sync_copy(rows.at[pl.ds(0, tail)],
                            part_hbm.at[c, pl.ds(tb, tail)])

    return ek(es_flat, ed_flat, h, src3, dst3)


def kernel(x, edge_index, W1_src, W1_dst, a1_src, a1_dst, b1, Wl1, bl1,
           W2_src, W2_dst, a2_src, a2_dst, b2, Wl2, bl2):
    n, d = x.shape
    e = edge_index.shape[1]
    per = e // NW
    wn = pl.cdiv(per, WIN)
    perp = wn * WIN

    src = edge_index[0].reshape(NW, per)
    dst = edge_index[1].reshape(NW, per)
    pad = jnp.zeros((NW, perp - per), jnp.int32)
    src3 = jnp.concatenate([src, pad], axis=1).reshape(NW, wn, WIN)
    dst3 = jnp.concatenate([dst, pad], axis=1).reshape(NW, wn, WIN)

    bm = 512

    h1, skip1, es1, ed1 = _tc_prep(x, W1_src, Wl1, bl1, a1_src, a1_dst,
                                   W1_dst, bm)
    part1 = _sc_edge(es1.reshape(n), ed1.reshape(n), h1, src3, dst3, n, per, d)
    h2, skip2, es2, ed2 = _tc_fuse_prep(part1, b1, skip1, W2_src, Wl2, bl2,
                                        a2_src, a2_dst, W2_dst, bm)
    part2 = _sc_edge(es2.reshape(n), ed2.reshape(n), h2, src3, dst3, n, per, d)
    return _tc_final(part2, b2, skip2, bm)


# speedup vs baseline: 1.4809x; 1.0031x over previous
import functools

import jax
import jax.numpy as jnp
from jax import lax
from jax.experimental import pallas as pl
from jax.experimental.pallas import tpu as pltpu
from jax.experimental.pallas import tpu_sc as plsc

NC = 2
NS = 16
NW = NC * NS
WIN = 128


def _tc_prep(xin, W_src, Wl, bl, a_src, a_dst, W_dst, bm):
    n, d = xin.shape

    def body(x_ref, ws_ref, wl_ref, bl_ref, as_ref, ad_ref, wd_ref,
             h_ref, sk_ref, es_ref, ed_ref):
        xb = x_ref[...]
        h_ref[...] = jnp.dot(xb, ws_ref[...], preferred_element_type=jnp.float32)
        sk_ref[...] = jnp.dot(xb, wl_ref[...], preferred_element_type=jnp.float32) + bl_ref[...]
        ws2 = jnp.sum(ws_ref[...] * as_ref[...], axis=1, keepdims=True)
        wd2 = jnp.sum(wd_ref[...] * ad_ref[...], axis=1, keepdims=True)
        es_ref[...] = jnp.dot(xb, ws2, preferred_element_type=jnp.float32)
        ed_ref[...] = jnp.dot(xb, wd2, preferred_element_type=jnp.float32)

    full = lambda i: (0, 0)
    return pl.pallas_call(
        body,
        grid=(pl.cdiv(n, bm),),
        in_specs=[pl.BlockSpec((bm, d), lambda i: (i, 0)),
                  pl.BlockSpec((d, d), full),
                  pl.BlockSpec((d, d), full),
                  pl.BlockSpec((1, d), full),
                  pl.BlockSpec((1, d), full),
                  pl.BlockSpec((1, d), full),
                  pl.BlockSpec((d, d), full)],
        out_specs=[pl.BlockSpec((bm, d), lambda i: (i, 0)),
                   pl.BlockSpec((bm, d), lambda i: (i, 0)),
                   pl.BlockSpec((bm, 1), lambda i: (i, 0)),
                   pl.BlockSpec((bm, 1), lambda i: (i, 0))],
        out_shape=[jax.ShapeDtypeStruct((n, d), jnp.float32),
                   jax.ShapeDtypeStruct((n, d), jnp.float32),
                   jax.ShapeDtypeStruct((n, 1), jnp.float32),
                   jax.ShapeDtypeStruct((n, 1), jnp.float32)],
    )(xin, W_src, Wl, bl.reshape(1, d), a_src.reshape(1, d),
      a_dst.reshape(1, d), W_dst)


def _tc_fuse_prep(part, b, skip, W_src, Wl, bl, a_src, a_dst, W_dst, bm):
    _, n, d = part.shape

    def body(p_ref, b_ref, skA_ref, ws_ref, wl_ref, bl_ref, as_ref, ad_ref,
             wd_ref, h2_ref, sk_ref, es_ref, ed_ref):
        hb = jnp.maximum(p_ref[0] + p_ref[1] + b_ref[...] + skA_ref[...], 0.0)
        h2_ref[...] = jnp.dot(hb, ws_ref[...], preferred_element_type=jnp.float32)
        sk_ref[...] = jnp.dot(hb, wl_ref[...], preferred_element_type=jnp.float32) + bl_ref[...]
        ws2 = jnp.sum(ws_ref[...] * as_ref[...], axis=1, keepdims=True)
        wd2 = jnp.sum(wd_ref[...] * ad_ref[...], axis=1, keepdims=True)
        es_ref[...] = jnp.dot(hb, ws2, preferred_element_type=jnp.float32)
        ed_ref[...] = jnp.dot(hb, wd2, preferred_element_type=jnp.float32)

    full = lambda i: (0, 0)
    return pl.pallas_call(
        body,
        grid=(pl.cdiv(n, bm),),
        in_specs=[pl.BlockSpec((2, bm, d), lambda i: (0, i, 0)),
                  pl.BlockSpec((1, d), full),
                  pl.BlockSpec((bm, d), lambda i: (i, 0)),
                  pl.BlockSpec((d, d), full),
                  pl.BlockSpec((d, d), full),
                  pl.BlockSpec((1, d), full),
                  pl.BlockSpec((1, d), full),
                  pl.BlockSpec((1, d), full),
                  pl.BlockSpec((d, d), full)],
        out_specs=[pl.BlockSpec((bm, d), lambda i: (i, 0)),
                   pl.BlockSpec((bm, d), lambda i: (i, 0)),
                   pl.BlockSpec((bm, 1), lambda i: (i, 0)),
                   pl.BlockSpec((bm, 1), lambda i: (i, 0))],
        out_shape=[jax.ShapeDtypeStruct((n, d), jnp.float32),
                   jax.ShapeDtypeStruct((n, d), jnp.float32),
                   jax.ShapeDtypeStruct((n, 1), jnp.float32),
                   jax.ShapeDtypeStruct((n, 1), jnp.float32)],
    )(part, b.reshape(1, d), skip, W_src, Wl, bl.reshape(1, d),
      a_src.reshape(1, d), a_dst.reshape(1, d), W_dst)


def _tc_final(part, b, skip, bm):
    _, n, d = part.shape

    def body(p_ref, b_ref, sk_ref, o_ref):
        o_ref[...] = p_ref[0] + p_ref[1] + b_ref[...] + sk_ref[...]

    full = lambda i: (0, 0)
    return pl.pallas_call(
        body,
        grid=(pl.cdiv(n, bm),),
        in_specs=[pl.BlockSpec((2, bm, d), lambda i: (0, i, 0)),
                  pl.BlockSpec((1, d), full),
                  pl.BlockSpec((bm, d), lambda i: (i, 0))],
        out_specs=pl.BlockSpec((bm, d), lambda i: (i, 0)),
        out_shape=jax.ShapeDtypeStruct((n, d), jnp.float32),
    )(part, b.reshape(1, d), skip)


def _sc_edge(es_flat, ed_flat, h, src3, dst3, n_nodes, per, d):
    wn = src3.shape[1]
    npt = ((n_nodes // NS) // 8) * 8
    tail = n_nodes - NS * npt
    assert tail <= WIN and npt % 16 == 0 and tail % 16 == 0
    chunks = [WIN] * (npt // WIN) + ([npt % WIN] if npt % WIN else [])

    mesh = plsc.VectorSubcoreMesh(core_axis_name="c", subcore_axis_name="s",
                                  num_cores=NC, num_subcores=NS)

    @functools.partial(
        pl.kernel,
        out_type=jax.ShapeDtypeStruct((NC, n_nodes, d), jnp.float32),
        mesh=mesh,
        compiler_params=pltpu.CompilerParams(needs_layout_passes=False),
        scratch_types=[
            pltpu.VMEM((wn, WIN), jnp.int32),
            pltpu.VMEM((wn, WIN), jnp.int32),
            pltpu.VMEM((wn, WIN), jnp.float32),
            pltpu.VMEM((WIN,), jnp.float32),
            pltpu.VMEM((WIN,), jnp.float32),
            pltpu.VMEM((WIN,), jnp.float32),
            pltpu.VMEM((npt + 16,), jnp.float32),
            pltpu.VMEM((WIN, d), jnp.float32),
            pltpu.VMEM_SHARED((n_nodes,), jnp.float32),
            pltpu.VMEM_SHARED((n_nodes,), jnp.float32),
            pltpu.VMEM_SHARED((n_nodes,), jnp.float32),
            pltpu.VMEM_SHARED((n_nodes, d), jnp.float32),
            pltpu.SemaphoreType.DMA,
            pltpu.SemaphoreType.DMA,
            pltpu.SemaphoreType.DMA,
        ],
    )
    def ek(es_hbm, ed_hbm, h_hbm, src_hbm, dst_hbm, part_hbm,
           srcO, dstO, ex2d, esg, edg, exw, invb, rows,
           es_sh, ed_sh, s_sh, out_sh, sem, semB, semS):
        c = lax.axis_index("c")
        s = lax.axis_index("s")
        own = c * NS + s
        mir = (1 - c) * NS + s
        nb = s * npt
        last = NS - 1
        zv = jnp.zeros((16,), jnp.float32)

        pltpu.sync_copy(src_hbm.at[own], srcO)
        pltpu.sync_copy(dst_hbm.at[own], dstO)

        pltpu.sync_copy(es_hbm.at[pl.ds(nb, npt)], invb.at[pl.ds(0, npt)])
        pltpu.sync_copy(invb.at[pl.ds(0, npt)], es_sh.at[pl.ds(nb, npt)])
        pltpu.sync_copy(ed_hbm.at[pl.ds(nb, npt)], invb.at[pl.ds(0, npt)])
        pltpu.sync_copy(invb.at[pl.ds(0, npt)], ed_sh.at[pl.ds(nb, npt)])

        @pl.when(s == last)
        def _():
            tb = NS * npt
            pltpu.sync_copy(es_hbm.at[pl.ds(tb, tail)], exw.at[pl.ds(0, tail)])
            pltpu.sync_copy(exw.at[pl.ds(0, tail)], es_sh.at[pl.ds(tb, tail)])
            pltpu.sync_copy(ed_hbm.at[pl.ds(tb, tail)], exw.at[pl.ds(0, tail)])
            pltpu.sync_copy(exw.at[pl.ds(0, tail)], ed_sh.at[pl.ds(tb, tail)])

        def zs(i, _):
            invb[pl.ds(16 * i, 16)] = zv
            return 0
        lax.fori_loop(0, (npt + 16) // 16, zs, 0)
        pltpu.sync_copy(invb.at[pl.ds(0, npt)], s_sh.at[pl.ds(nb, npt)])

        @pl.when(s == last)
        def _():
            pltpu.sync_copy(invb.at[pl.ds(0, tail)],
                            s_sh.at[pl.ds(NS * npt, tail)])

        def zrow(i, _):
            for j in range(d // 16):
                rows[i, pl.ds(16 * j, 16)] = zv
            return 0
        lax.fori_loop(0, WIN, zrow, 0)
        off = 0
        for cl in chunks:
            pltpu.sync_copy(rows.at[pl.ds(0, cl)],
                            out_sh.at[pl.ds(nb + off, cl)])
            off += cl

        @pl.when(s == last)
        def _():
            pltpu.sync_copy(rows.at[pl.ds(0, tail)],
                            out_sh.at[pl.ds(NS * npt, tail)])

        plsc.subcore_barrier()

        def exp_win(w, out_ref, keep2d):
            for v in range(WIN // 16):
                e = esg[pl.ds(16 * v, 16)] + edg[pl.ds(16 * v, 16)]
                e = jnp.where(e > 0, e, 0.2 * e)
                e = jnp.minimum(e, 70.0)
                ex = jnp.exp(e)
                pos = w * WIN + 16 * v + lax.iota(jnp.int32, 16)
                ex = jnp.where(pos < per, ex, 0.0)
                if keep2d:
                    out_ref[w, pl.ds(16 * v, 16)] = ex
                else:
                    out_ref[pl.ds(16 * v, 16)] = ex

        def drain_scalar():
            pltpu.make_async_copy(es_hbm.at[pl.ds(0, WIN)], exw, semS).wait()

        def drain_rows():
            pltpu.make_async_copy(h_hbm.at[pl.ds(0, WIN)], rows, semS).wait()

        def win1(w, _):
            ga = pltpu.async_copy(es_sh.at[srcO.at[w]], esg, sem)
            gb = pltpu.async_copy(ed_sh.at[dstO.at[w]], edg, semB)
            ga.wait()
            gb.wait()
            exp_win(w, ex2d, True)

            @pl.when(w > 0)
            def _():
                drain_scalar()
            pltpu.async_copy(ex2d.at[w], s_sh.at[dstO.at[w]], semS, add=True)
            return 0
        lax.fori_loop(0, wn, win1, 0)
        drain_scalar()

        pltpu.sync_copy(src_hbm.at[mir], srcO)
        pltpu.sync_copy(dst_hbm.at[mir], dstO)

        def win1m(w, _):
            ga = pltpu.async_copy(es_sh.at[srcO.at[w]], esg, sem)
            gb = pltpu.async_copy(ed_sh.at[dstO.at[w]], edg, semB)
            ga.wait()
            gb.wait()

            @pl.when(w > 0)
            def _():
                drain_scalar()
            exp_win(w, exw, False)
            pltpu.async_copy(exw, s_sh.at[dstO.at[w]], semS, add=True)
            return 0
        lax.fori_loop(0, wn, win1m, 0)
        drain_scalar()

        plsc.subcore_barrier()

        pltpu.sync_copy(s_sh.at[pl.ds(nb, npt)], invb.at[pl.ds(0, npt)])

        def inv(i, _):
            sv = invb[pl.ds(16 * i, 16)]
            invb[pl.ds(16 * i, 16)] = 1.0 / (sv + 1e-16)
            return 0
        lax.fori_loop(0, npt // 16, inv, 0)
        pltpu.sync_copy(invb.at[pl.ds(0, npt)], s_sh.at[pl.ds(nb, npt)])

        @pl.when(s == last)
        def _():
            tb = NS * npt
            pltpu.sync_copy(s_sh.at[pl.ds(tb, tail)], invb.at[pl.ds(0, tail)])
            sv = invb[pl.ds(0, 16)]
            invb[pl.ds(0, 16)] = 1.0 / (sv + 1e-16)
            pltpu.sync_copy(invb.at[pl.ds(0, tail)], s_sh.at[pl.ds(tb, tail)])

        pltpu.sync_copy(src_hbm.at[own], srcO)
        pltpu.sync_copy(dst_hbm.at[own], dstO)
        plsc.subcore_barrier()

        def win2(w, _):
            pltpu.sync_copy(s_sh.at[dstO.at[w]], edg)
            for v in range(WIN // 16):
                edg[pl.ds(16 * v, 16)] = (ex2d[w, pl.ds(16 * v, 16)]
                                          * edg[pl.ds(16 * v, 16)])

            @pl.when(w > 0)
            def _():
                drain_rows()
            pltpu.async_copy(h_hbm.at[srcO.at[w]], rows, sem).wait()

            def scale_v(v, _):
                av = edg[pl.ds(16 * v, 16)]
                for t in range(16):
                    a = av[t]
                    i = 16 * v + t
                    for j in range(d // 16):
                        rows[i, pl.ds(16 * j, 16)] = rows[i, pl.ds(16 * j, 16)] * a
                return 0
            lax.fori_loop(0, WIN // 16, scale_v, 0)
            pltpu.async_copy(rows, out_sh.at[dstO.at[w]], semS, add=True)
            return 0
        lax.fori_loop(0, wn, win2, 0)
        drain_rows()

        plsc.subcore_barrier()

        off = 0
        for cl in chunks:
            pltpu.sync_copy(out_sh.at[pl.ds(nb + off, cl)],
                            rows.at[pl.ds(0, cl)])
            pltpu.sync_copy(rows.at[pl.ds(0, cl)],
                            part_hbm.at[c, pl.ds(nb + off, cl)])
            off += cl

        @pl.when(s == last)
        def _():
            tb = NS * npt
            pltpu.sync_copy(out_sh.at[pl.ds(tb, tail)], rows.at[pl.ds(0, tail)])
            pltpu.sync_copy(rows.at[pl.ds(0, tail)],
                            part_hbm.at[c, pl.ds(tb, tail)])

    return ek(es_flat, ed_flat, h, src3, dst3)


def kernel(x, edge_index, W1_src, W1_dst, a1_src, a1_dst, b1, Wl1, bl1,
           W2_src, W2_dst, a2_src, a2_dst, b2, Wl2, bl2):
    n, d = x.shape
    e = edge_index.shape[1]
    per = e // NW
    wn = pl.cdiv(per, WIN)
    perp = wn * WIN

    src = edge_index[0].reshape(NW, per)
    dst = edge_index[1].reshape(NW, per)
    pad = jnp.zeros((NW, perp - per), jnp.int32)
    src3 = jnp.concatenate([src, pad], axis=1).reshape(NW, wn, WIN)
    dst3 = jnp.concatenate([dst, pad], axis=1).reshape(NW, wn, WIN)

    bm = 512

    h1, skip1, es1, ed1 = _tc_prep(x, W1_src, Wl1, bl1, a1_src, a1_dst,
                                   W1_dst, bm)
    part1 = _sc_edge(es1.reshape(n), ed1.reshape(n), h1, src3, dst3, n, per, d)
    h2, skip2, es2, ed2 = _tc_fuse_prep(part1, b1, skip1, W2_src, Wl2, bl2,
                                        a2_src, a2_dst, W2_dst, bm)
    part2 = _sc_edge(es2.reshape(n), ed2.reshape(n), h2, src3, dst3, n, per, d)
    return _tc_final(part2, b2, skip2, bm)
